# speedup vs baseline: 13.0692x; 13.0692x over previous
import functools

import jax
import jax.numpy as jnp
from jax import lax
from jax.experimental import pallas as pl
from jax.experimental.pallas import tpu as pltpu
from jax.experimental.pallas import tpu_sc as plsc

N_NODES = 10000
F = 128
N_EDGES = 320000

NC = 2
NS = 16
NW = NC * NS

CHUNK = 128
CHUNKS_PER_W = 79
EPAD = NW * CHUNKS_PER_W * CHUNK
NPAD = 10240
ROWS_PER_SUB = NPAD // NS

_mesh = plsc.VectorSubcoreMesh(core_axis_name="c", subcore_axis_name="s")


@functools.partial(
    pl.kernel,
    mesh=_mesh,
    out_type=jax.ShapeDtypeStruct((NC, NPAD, 16), jnp.float32),
    scratch_types=[
        pltpu.VMEM((CHUNKS_PER_W, CHUNK), jnp.int32),
        pltpu.VMEM((CHUNK, 16), jnp.float32),
        pltpu.VMEM((16, 16), jnp.float32),
        pltpu.VMEM_SHARED((NPAD, 16), jnp.float32),
    ],
)
def _sc_degree(dst_hbm, out_hbm, dstv, ones, zb, acc):
    c = lax.axis_index("c")
    s = lax.axis_index("s")
    wid = c * NS + s

    @pl.loop(0, CHUNK)
    def _(r):
        ones[r, pl.ds(0, 16)] = jnp.ones((16,), jnp.float32)

    @pl.loop(0, 16)
    def _(r):
        zb[r, pl.ds(0, 16)] = jnp.zeros((16,), jnp.float32)

    @pl.loop(0, ROWS_PER_SUB // 16)
    def _(t):
        pltpu.sync_copy(zb, acc.at[pl.ds(s * ROWS_PER_SUB + t * 16, 16)])

    pltpu.sync_copy(dst_hbm.at[wid], dstv)
    plsc.subcore_barrier()

    @pl.loop(0, CHUNKS_PER_W)
    def _(j):
        pltpu.sync_copy(ones, acc.at[dstv.at[j]], add=True)

    plsc.subcore_barrier()
    pltpu.sync_copy(
        acc.at[pl.ds(s * ROWS_PER_SUB, ROWS_PER_SUB)],
        out_hbm.at[c, pl.ds(s * ROWS_PER_SUB, ROWS_PER_SUB)],
    )


@functools.partial(
    pl.kernel,
    mesh=_mesh,
    out_type=jax.ShapeDtypeStruct((NC, NPAD, F), jnp.float32),
    scratch_types=[
        pltpu.VMEM((CHUNKS_PER_W, CHUNK), jnp.int32),
        pltpu.VMEM((CHUNKS_PER_W, CHUNK), jnp.int32),
        pltpu.VMEM((CHUNK, F), jnp.float32),
        pltpu.VMEM((16, F), jnp.float32),
        pltpu.VMEM_SHARED((NPAD, F), jnp.float32),
        pltpu.SemaphoreType.DMA,
    ],
)
def _sc_propagate(hp_hbm, src_hbm, dst_hbm, out_hbm, srcv, dstv, rows, zb, acc, sem):
    c = lax.axis_index("c")
    s = lax.axis_index("s")
    wid = c * NS + s

    @pl.loop(0, 16)
    def _(r):
        @pl.loop(0, F // 16)
        def _(g):
            zb[r, pl.ds(g * 16, 16)] = jnp.zeros((16,), jnp.float32)

    @pl.loop(0, ROWS_PER_SUB // 16)
    def _(t):
        pltpu.sync_copy(zb, acc.at[pl.ds(s * ROWS_PER_SUB + t * 16, 16)])

    pltpu.sync_copy(src_hbm.at[wid], srcv)
    pltpu.sync_copy(dst_hbm.at[wid], dstv)
    plsc.subcore_barrier()

    @pl.loop(0, CHUNKS_PER_W)
    def _(j):
        pltpu.async_copy(hp_hbm.at[srcv.at[j]], rows, sem).wait()
        pltpu.sync_copy(rows, acc.at[dstv.at[j]], add=True)

    plsc.subcore_barrier()
    pltpu.sync_copy(
        acc.at[pl.ds(s * ROWS_PER_SUB, ROWS_PER_SUB)],
        out_hbm.at[c, pl.ds(s * ROWS_PER_SUB, ROWS_PER_SUB)],
    )


_BR = 1024


def _dinv_block(degp):
    deg = degp[0, :, 0] + degp[1, :, 0] + 1.0
    return lax.rsqrt(deg)[:, None]


def _tc_first(x_ref, w_ref, degp_ref, o_ref):
    dinv = _dinv_block(degp_ref[...])
    h = jnp.dot(x_ref[...], w_ref[...], preferred_element_type=jnp.float32)
    o_ref[...] = h * dinv


def _tc_mid(p_ref, hp_ref, degp_ref, b_ref, w_ref, o_ref):
    dinv = _dinv_block(degp_ref[...])
    t = dinv * (p_ref[0] + p_ref[1] + hp_ref[...]) + b_ref[...]
    a = jnp.where(t >= 0.0, t, 0.2 * t)
    h = jnp.dot(a, w_ref[...], preferred_element_type=jnp.float32)
    o_ref[...] = h * dinv


def _tc_last(p_ref, hp_ref, degp_ref, b_ref, o_ref):
    dinv = _dinv_block(degp_ref[...])
    t = dinv * (p_ref[0] + p_ref[1] + hp_ref[...]) + b_ref[...]
    m = jnp.max(t, axis=1, keepdims=True)
    e = jnp.exp(t - m)
    lse = jnp.log(jnp.sum(e, axis=1, keepdims=True))
    o_ref[...] = (t - m) - lse


_row_spec = pl.BlockSpec((_BR, F), lambda i: (i, 0))
_part_spec = pl.BlockSpec((NC, _BR, F), lambda i: (0, i, 0))
_degp_spec = pl.BlockSpec((NC, _BR, 16), lambda i: (0, i, 0))
_w_spec = pl.BlockSpec((F, F), lambda i: (0, 0))
_b_spec = pl.BlockSpec((1, F), lambda i: (0, 0))
_grid = (NPAD // _BR,)
_out_rows = jax.ShapeDtypeStruct((NPAD, F), jnp.float32)


def kernel(x, edge_index, W1, b1, W2, b2):
    src = edge_index[0].astype(jnp.int32)
    dst = edge_index[1].astype(jnp.int32)
    pad = jnp.full((EPAD - N_EDGES,), NPAD - 1, jnp.int32)
    src3 = jnp.concatenate([src, pad]).reshape(NW, CHUNKS_PER_W, CHUNK)
    dst3 = jnp.concatenate([dst, pad]).reshape(NW, CHUNKS_PER_W, CHUNK)
    xpad = jnp.zeros((NPAD, F), jnp.float32).at[:N_NODES].set(x)
    b1r = b1.reshape(1, F)
    b2r = b2.reshape(1, F)

    degp = _sc_degree(dst3)

    h1p = pl.pallas_call(
        _tc_first,
        grid=_grid,
        in_specs=[_row_spec, _w_spec, _degp_spec],
        out_specs=_row_spec,
        out_shape=_out_rows,
    )(xpad, W1, degp)

    p1 = _sc_propagate(h1p, src3, dst3)

    h2p = pl.pallas_call(
        _tc_mid,
        grid=_grid,
        in_specs=[_part_spec, _row_spec, _degp_spec, _b_spec, _w_spec],
        out_specs=_row_spec,
        out_shape=_out_rows,
    )(p1, h1p, degp, b1r, W2)

    p2 = _sc_propagate(h2p, src3, dst3)

    out = pl.pallas_call(
        _tc_last,
        grid=_grid,
        in_specs=[_part_spec, _row_spec, _degp_spec, _b_spec],
        out_specs=_row_spec,
        out_shape=_out_rows,
    )(p2, h2p, degp, b2r)

    return out[:N_NODES]


# speedup vs baseline: 22.1105x; 1.6918x over previous
import functools

import jax
import jax.numpy as jnp
from jax import lax
from jax.experimental import pallas as pl
from jax.experimental.pallas import tpu as pltpu
from jax.experimental.pallas import tpu_sc as plsc

N_NODES = 10000
F = 128
N_EDGES = 320000

NC = 2
NS = 16
NW = NC * NS

CHUNK = 128
CHUNKS_PER_W = 79
EPAD = NW * CHUNKS_PER_W * CHUNK
NPAD = 10240
ROWS_PER_SUB = NPAD // NS

_mesh = plsc.VectorSubcoreMesh(core_axis_name="c", subcore_axis_name="s")


@functools.partial(
    pl.kernel,
    mesh=_mesh,
    out_type=jax.ShapeDtypeStruct((NC, NPAD, 16), jnp.float32),
    scratch_types=[
        pltpu.VMEM((CHUNKS_PER_W, CHUNK), jnp.int32),
        pltpu.VMEM((CHUNK, 16), jnp.float32),
        pltpu.VMEM((16, 16), jnp.float32),
        pltpu.VMEM_SHARED((NPAD, 16), jnp.float32),
    ],
)
def _sc_degree(dst_hbm, out_hbm, dstv, ones, zb, acc):
    c = lax.axis_index("c")
    s = lax.axis_index("s")
    wid = c * NS + s

    @pl.loop(0, CHUNK)
    def _(r):
        ones[r, pl.ds(0, 16)] = jnp.ones((16,), jnp.float32)

    @pl.loop(0, 16)
    def _(r):
        zb[r, pl.ds(0, 16)] = jnp.zeros((16,), jnp.float32)

    @pl.loop(0, ROWS_PER_SUB // 16)
    def _(t):
        pltpu.sync_copy(zb, acc.at[pl.ds(s * ROWS_PER_SUB + t * 16, 16)])

    pltpu.sync_copy(dst_hbm.at[wid], dstv)
    plsc.subcore_barrier()

    @pl.loop(0, CHUNKS_PER_W)
    def _(j):
        pltpu.sync_copy(ones, acc.at[dstv.at[j]], add=True)

    plsc.subcore_barrier()
    pltpu.sync_copy(
        acc.at[pl.ds(s * ROWS_PER_SUB, ROWS_PER_SUB)],
        out_hbm.at[c, pl.ds(s * ROWS_PER_SUB, ROWS_PER_SUB)],
    )


@functools.partial(
    pl.kernel,
    mesh=_mesh,
    out_type=jax.ShapeDtypeStruct((NC, NPAD, F), jnp.float32),
    scratch_types=[
        pltpu.VMEM((CHUNKS_PER_W, CHUNK), jnp.int32),
        pltpu.VMEM((CHUNKS_PER_W, CHUNK), jnp.int32),
        pltpu.VMEM((CHUNK, F), jnp.float32),
        pltpu.VMEM((16, F), jnp.float32),
        pltpu.VMEM_SHARED((NPAD, F), jnp.float32),
        pltpu.SemaphoreType.DMA,
    ],
)
def _sc_propagate(hp_hbm, src_hbm, dst_hbm, out_hbm, srcv, dstv, rows, zb, acc, sem):
    c = lax.axis_index("c")
    s = lax.axis_index("s")
    wid = c * NS + s

    @pl.loop(0, 16)
    def _(r):
        @pl.loop(0, F // 16)
        def _(g):
            zb[r, pl.ds(g * 16, 16)] = jnp.zeros((16,), jnp.float32)

    @pl.loop(0, ROWS_PER_SUB // 16)
    def _(t):
        pltpu.sync_copy(zb, acc.at[pl.ds(s * ROWS_PER_SUB + t * 16, 16)])

    pltpu.sync_copy(src_hbm.at[wid], srcv)
    pltpu.sync_copy(dst_hbm.at[wid], dstv)
    plsc.subcore_barrier()

    @pl.loop(0, CHUNKS_PER_W)
    def _(j):
        pltpu.async_copy(hp_hbm.at[srcv.at[j]], rows, sem).wait()
        pltpu.sync_copy(rows, acc.at[dstv.at[j]], add=True)

    plsc.subcore_barrier()
    pltpu.sync_copy(
        acc.at[pl.ds(s * ROWS_PER_SUB, ROWS_PER_SUB)],
        out_hbm.at[c, pl.ds(s * ROWS_PER_SUB, ROWS_PER_SUB)],
    )


_BR = 1024


def _dinv_block(degp):
    deg = degp[0, :, 0] + degp[1, :, 0] + 1.0
    return lax.rsqrt(deg)[:, None]


def _tc_first(x_ref, w_ref, degp_ref, o_ref):
    dinv = _dinv_block(degp_ref[...])
    h = jnp.dot(x_ref[...], w_ref[...], preferred_element_type=jnp.float32)
    o_ref[...] = h * dinv


def _tc_mid(p_ref, hp_ref, degp_ref, b_ref, w_ref, o_ref):
    dinv = _dinv_block(degp_ref[...])
    t = dinv * (p_ref[0] + p_ref[1] + hp_ref[...]) + b_ref[...]
    a = jnp.where(t >= 0.0, t, 0.2 * t)
    h = jnp.dot(a, w_ref[...], preferred_element_type=jnp.float32)
    o_ref[...] = h * dinv


def _tc_last(p_ref, hp_ref, degp_ref, b_ref, o_ref):
    dinv = _dinv_block(degp_ref[...])
    t = dinv * (p_ref[0] + p_ref[1] + hp_ref[...]) + b_ref[...]
    m = jnp.max(t, axis=1, keepdims=True)
    e = jnp.exp(t - m)
    lse = jnp.log(jnp.sum(e, axis=1, keepdims=True))
    o_ref[...] = (t - m) - lse


_row_spec = pl.BlockSpec((_BR, F), lambda i: (i, 0))
_part_spec = pl.BlockSpec((NC, _BR, F), lambda i: (0, i, 0))
_degp_spec = pl.BlockSpec((NC, _BR, 16), lambda i: (0, i, 0))
_w_spec = pl.BlockSpec((F, F), lambda i: (0, 0))
_b_spec = pl.BlockSpec((1, F), lambda i: (0, 0))
_grid = (NPAD // _BR,)
_out_rows = jax.ShapeDtypeStruct((NPAD, F), jnp.float32)


def kernel(x, edge_index, W1, b1, W2, b2):
    src = edge_index[0].astype(jnp.int32)
    dst = edge_index[1].astype(jnp.int32)
    e_per_w = N_EDGES // NW
    pad_per_w = CHUNKS_PER_W * CHUNK - e_per_w
    padv = (N_NODES + (jnp.arange(NW * pad_per_w, dtype=jnp.int32)
                       % (NPAD - N_NODES))).reshape(NW, pad_per_w)
    src3 = jnp.concatenate([src.reshape(NW, e_per_w), padv],
                           axis=1).reshape(NW, CHUNKS_PER_W, CHUNK)
    dst3 = jnp.concatenate([dst.reshape(NW, e_per_w), padv],
                           axis=1).reshape(NW, CHUNKS_PER_W, CHUNK)
    xpad = jnp.zeros((NPAD, F), jnp.float32).at[:N_NODES].set(x)
    b1r = b1.reshape(1, F)
    b2r = b2.reshape(1, F)

    degp = _sc_degree(dst3)

    h1p = pl.pallas_call(
        _tc_first,
        grid=_grid,
        in_specs=[_row_spec, _w_spec, _degp_spec],
        out_specs=_row_spec,
        out_shape=_out_rows,
    )(xpad, W1, degp)

    p1 = _sc_propagate(h1p, src3, dst3)

    h2p = pl.pallas_call(
        _tc_mid,
        grid=_grid,
        in_specs=[_part_spec, _row_spec, _degp_spec, _b_spec, _w_spec],
        out_specs=_row_spec,
        out_shape=_out_rows,
    )(p1, h1p, degp, b1r, W2)

    p2 = _sc_propagate(h2p, src3, dst3)

    out = pl.pallas_call(
        _tc_last,
        grid=_grid,
        in_specs=[_part_spec, _row_spec, _degp_spec, _b_spec],
        out_specs=_row_spec,
        out_shape=_out_rows,
    )(p2, h2p, degp, b2r)

    return out[:N_NODES]


# speedup vs baseline: 31.0537x; 1.4045x over previous
import functools

import jax
import jax.numpy as jnp
from jax import lax
from jax.experimental import pallas as pl
from jax.experimental.pallas import tpu as pltpu
from jax.experimental.pallas import tpu_sc as plsc

N_NODES = 10000
F = 128
N_EDGES = 320000

NC = 2
NS = 16
NW = NC * NS

CHUNK = 128
CHUNKS_PER_W = 80
HALVES = 2
HCH = CHUNKS_PER_W // HALVES
EPAD = NW * CHUNKS_PER_W * CHUNK
NPAD = 10240
ROWS_PER_SUB = NPAD // NS

_mesh = plsc.VectorSubcoreMesh(core_axis_name="c", subcore_axis_name="s")


@functools.partial(
    pl.kernel,
    mesh=_mesh,
    out_type=jax.ShapeDtypeStruct((NC, NPAD, 16), jnp.float32),
    scratch_types=[
        pltpu.VMEM((HALVES, HCH, CHUNK), jnp.int32),
        pltpu.VMEM((CHUNK, 16), jnp.float32),
        pltpu.VMEM((16, 16), jnp.float32),
        pltpu.VMEM_SHARED((NPAD, 16), jnp.float32),
    ],
)
def _sc_degree(dst_hbm, out_hbm, dstv, ones, zb, acc):
    c = lax.axis_index("c")
    s = lax.axis_index("s")
    wid = c * NS + s

    @pl.loop(0, CHUNK)
    def _(r):
        ones[r, pl.ds(0, 16)] = jnp.ones((16,), jnp.float32)

    @pl.loop(0, 16)
    def _(r):
        zb[r, pl.ds(0, 16)] = jnp.zeros((16,), jnp.float32)

    @pl.loop(0, ROWS_PER_SUB // 16)
    def _(t):
        pltpu.sync_copy(zb, acc.at[pl.ds(s * ROWS_PER_SUB + t * 16, 16)])

    pltpu.sync_copy(dst_hbm.at[wid], dstv)
    plsc.subcore_barrier()

    @pl.loop(0, HALVES)
    def _(h):
        @pl.loop(0, HCH)
        def _(j):
            pltpu.sync_copy(ones, acc.at[dstv.at[h, j]], add=True)

    plsc.subcore_barrier()
    pltpu.sync_copy(
        acc.at[pl.ds(s * ROWS_PER_SUB, ROWS_PER_SUB)],
        out_hbm.at[c, pl.ds(s * ROWS_PER_SUB, ROWS_PER_SUB)],
    )


@functools.partial(
    pl.kernel,
    mesh=_mesh,
    out_type=jax.ShapeDtypeStruct((NC, NPAD, F), jnp.float32),
    scratch_types=[
        pltpu.VMEM((HCH, CHUNK), jnp.int32),
        pltpu.VMEM((HCH, CHUNK), jnp.int32),
        pltpu.VMEM((CHUNK, F), jnp.float32),
        pltpu.VMEM((CHUNK, F), jnp.float32),
        pltpu.VMEM((16, F), jnp.float32),
        pltpu.VMEM_SHARED((NPAD, F), jnp.float32),
        pltpu.SemaphoreType.DMA,
        pltpu.SemaphoreType.DMA,
    ],
)
def _sc_propagate(hp_hbm, src_hbm, dst_hbm, out_hbm, srcv, dstv, rows0, rows1,
                  zb, acc, sem0, sem1):
    c = lax.axis_index("c")
    s = lax.axis_index("s")
    wid = c * NS + s

    @pl.loop(0, 16)
    def _(r):
        @pl.loop(0, F // 16)
        def _(g):
            zb[r, pl.ds(g * 16, 16)] = jnp.zeros((16,), jnp.float32)

    @pl.loop(0, ROWS_PER_SUB // 16)
    def _(t):
        pltpu.sync_copy(zb, acc.at[pl.ds(s * ROWS_PER_SUB + t * 16, 16)])

    plsc.subcore_barrier()

    for h in range(HALVES):
        pltpu.sync_copy(src_hbm.at[wid, h], srcv)
        pltpu.sync_copy(dst_hbm.at[wid, h], dstv)

        pltpu.async_copy(hp_hbm.at[srcv.at[0]], rows0, sem0)

        @pl.loop(0, HCH, step=2)
        def _(t):
            pltpu.async_copy(hp_hbm.at[srcv.at[t + 1]], rows1, sem1)
            pltpu.make_async_copy(hp_hbm.at[srcv.at[t]], rows0, sem0).wait()
            pltpu.sync_copy(rows0, acc.at[dstv.at[t]], add=True)

            @pl.when(t + 2 < HCH)
            def _():
                pltpu.async_copy(hp_hbm.at[srcv.at[t + 2]], rows0, sem0)

            pltpu.make_async_copy(hp_hbm.at[srcv.at[t + 1]], rows1, sem1).wait()
            pltpu.sync_copy(rows1, acc.at[dstv.at[t + 1]], add=True)

    plsc.subcore_barrier()
    pltpu.sync_copy(
        acc.at[pl.ds(s * ROWS_PER_SUB, ROWS_PER_SUB)],
        out_hbm.at[c, pl.ds(s * ROWS_PER_SUB, ROWS_PER_SUB)],
    )


_BR = 1024


def _dinv_block(degp):
    deg = degp[0, :, 0] + degp[1, :, 0] + 1.0
    return lax.rsqrt(deg)[:, None]


def _tc_first(x_ref, w_ref, degp_ref, o_ref):
    dinv = _dinv_block(degp_ref[...])
    h = jnp.dot(x_ref[...], w_ref[...], preferred_element_type=jnp.float32)
    o_ref[...] = h * dinv


def _tc_mid(p_ref, hp_ref, degp_ref, b_ref, w_ref, o_ref):
    dinv = _dinv_block(degp_ref[...])
    t = dinv * (p_ref[0] + p_ref[1] + hp_ref[...]) + b_ref[...]
    a = jnp.where(t >= 0.0, t, 0.2 * t)
    h = jnp.dot(a, w_ref[...], preferred_element_type=jnp.float32)
    o_ref[...] = h * dinv


def _tc_last(p_ref, hp_ref, degp_ref, b_ref, o_ref):
    dinv = _dinv_block(degp_ref[...])
    t = dinv * (p_ref[0] + p_ref[1] + hp_ref[...]) + b_ref[...]
    m = jnp.max(t, axis=1, keepdims=True)
    e = jnp.exp(t - m)
    lse = jnp.log(jnp.sum(e, axis=1, keepdims=True))
    o_ref[...] = (t - m) - lse


_row_spec = pl.BlockSpec((_BR, F), lambda i: (i, 0))
_part_spec = pl.BlockSpec((NC, _BR, F), lambda i: (0, i, 0))
_degp_spec = pl.BlockSpec((NC, _BR, 16), lambda i: (0, i, 0))
_w_spec = pl.BlockSpec((F, F), lambda i: (0, 0))
_b_spec = pl.BlockSpec((1, F), lambda i: (0, 0))
_grid = (NPAD // _BR,)
_out_rows = jax.ShapeDtypeStruct((NPAD, F), jnp.float32)


def kernel(x, edge_index, W1, b1, W2, b2):
    src = edge_index[0].astype(jnp.int32)
    dst = edge_index[1].astype(jnp.int32)
    e_per_w = N_EDGES // NW
    pad_per_w = CHUNKS_PER_W * CHUNK - e_per_w
    padv = (N_NODES + (jnp.arange(NW * pad_per_w, dtype=jnp.int32)
                       % (NPAD - N_NODES))).reshape(NW, pad_per_w)
    src3 = jnp.concatenate([src.reshape(NW, e_per_w), padv],
                           axis=1).reshape(NW, HALVES, HCH, CHUNK)
    dst3 = jnp.concatenate([dst.reshape(NW, e_per_w), padv],
                           axis=1).reshape(NW, HALVES, HCH, CHUNK)
    xpad = jnp.zeros((NPAD, F), jnp.float32).at[:N_NODES].set(x)
    b1r = b1.reshape(1, F)
    b2r = b2.reshape(1, F)

    degp = _sc_degree(dst3)

    h1p = pl.pallas_call(
        _tc_first,
        grid=_grid,
        in_specs=[_row_spec, _w_spec, _degp_spec],
        out_specs=_row_spec,
        out_shape=_out_rows,
    )(xpad, W1, degp)

    p1 = _sc_propagate(h1p, src3, dst3)

    h2p = pl.pallas_call(
        _tc_mid,
        grid=_grid,
        in_specs=[_part_spec, _row_spec, _degp_spec, _b_spec, _w_spec],
        out_specs=_row_spec,
        out_shape=_out_rows,
    )(p1, h1p, degp, b1r, W2)

    p2 = _sc_propagate(h2p, src3, dst3)

    out = pl.pallas_call(
        _tc_last,
        grid=_grid,
        in_specs=[_part_spec, _row_spec, _degp_spec, _b_spec],
        out_specs=_row_spec,
        out_shape=_out_rows,
    )(p2, h2p, degp, b2r)

    return out[:N_NODES]
